# Initial kernel scaffold; baseline (speedup 1.0000x reference)
import jax
import jax.numpy as jnp
from jax.experimental import pallas as pl


def kernel(x, edge_index, lin_q_W, lin_q_b, gat_W, gat_b, gcn_W, gcn_b, le_W1, le_b1, le_W2, le_W3, le_b3):
    raise NotImplementedError("write your pallas kernel here")


# speedup vs baseline: 1.0326x; 1.0326x over previous
import functools
import jax
import jax.numpy as jnp
import numpy as np
from jax.experimental import pallas as pl

_N = 10000
_D = 128
_K = 1000
_NEG = 0.2
_BLK = 400


def _mm_body(a_ref, b_ref, o_ref):
    o_ref[...] = jnp.dot(a_ref[...], b_ref[...],
                         preferred_element_type=jnp.float32)


def _mm(a, b):
    n, d = a.shape
    c = b.shape[1]
    grid = n // _BLK
    return pl.pallas_call(
        _mm_body,
        grid=(grid,),
        in_specs=[
            pl.BlockSpec((_BLK, d), lambda i: (i, 0)),
            pl.BlockSpec((d, c), lambda i: (0, 0)),
        ],
        out_specs=pl.BlockSpec((_BLK, c), lambda i: (i, 0)),
        out_shape=jax.ShapeDtypeStruct((n, c), jnp.float32),
    )(a, b)


def _emat_body(s_ref, as_ref, o_ref):
    k = pl.program_id(0)

    @pl.when(k == 0)
    def _():
        o_ref[...] = jnp.zeros_like(o_ref)

    o_ref[...] += jax.lax.dot_general(
        s_ref[...], as_ref[...], (((0,), (0,)), ((), ())),
        preferred_element_type=jnp.float32)

    @pl.when(k == pl.num_programs(0) - 1)
    def _():
        ii = jax.lax.broadcasted_iota(jnp.int32, (_K, _K), 0)
        jj = jax.lax.broadcasted_iota(jnp.int32, (_K, _K), 1)
        o_ref[...] = jnp.where(ii == jj, 1.0, o_ref[...])


def _emat(s, as_):
    n, k = s.shape
    grid = n // _BLK
    return pl.pallas_call(
        _emat_body,
        grid=(grid,),
        in_specs=[
            pl.BlockSpec((_BLK, k), lambda i: (i, 0)),
            pl.BlockSpec((_BLK, k), lambda i: (i, 0)),
        ],
        out_specs=pl.BlockSpec((k, k), lambda i: (0, 0)),
        out_shape=jax.ShapeDtypeStruct((k, k), jnp.float32),
    )(s, as_)


def kernel(x, edge_index, lin_q_W, lin_q_b, gat_W, gat_b, gcn_W, gcn_b,
           le_W1, le_b1, le_W2, le_W3, le_b3):
    loop = jnp.arange(_N, dtype=edge_index.dtype)
    row = jnp.concatenate([edge_index[0], loop])
    col = jnp.concatenate([edge_index[1], loop])
    ones_e = jnp.ones(row.shape[0], dtype=x.dtype)

    deg = jax.ops.segment_sum(ones_e, col, num_segments=_N)
    dinv = jnp.where(deg > 0, 1.0 / jnp.sqrt(deg), 0.0)
    norm = dinv[row] * dinv[col]
    xw = _mm(x, gcn_W)
    x_pool = jax.ops.segment_sum(norm[:, None] * xw[row], col,
                                 num_segments=_N) + gcn_b

    X_q = jax.ops.segment_max(x_pool[col], row, num_segments=_N)
    M_q = _mm(X_q, lin_q_W) + lin_q_b
    q1 = _mm(M_q, gat_W[:_D, None])[:, 0]
    q2 = _mm(x_pool, gat_W[_D:, None])[:, 0]

    s = q1[row] + q2[col] + gat_b
    s = jnp.where(s > 0, s, _NEG * s)
    smax = jax.ops.segment_max(s, row, num_segments=_N)
    ex = jnp.exp(s - smax[row])
    den = jax.ops.segment_sum(ex, row, num_segments=_N)
    score = ex / (den[row] + 1e-16)

    out = jax.ops.segment_sum(x[col] * score[:, None], row, num_segments=_N)

    abc = _mm(out, jnp.concatenate([le_W1, le_W2, le_W3], axis=1))
    a = abc[:, 0] + le_b1[0]
    b = abc[:, 1]
    c3 = abc[:, 2] + le_b3[0]
    agg = deg * a - jax.ops.segment_sum(b[row], col, num_segments=_N)
    fitness = jax.nn.sigmoid(agg + c3)

    fvals, perm = jax.lax.top_k(fitness, _K)
    x_new = out[perm] * fvals[:, None]

    in_perm = jnp.zeros((_N,), dtype=bool).at[perm].set(True)
    emask = in_perm[row].astype(score.dtype)
    n_idx = jnp.zeros((_N,), dtype=jnp.int32).at[perm].set(
        jnp.arange(_K, dtype=jnp.int32))
    scol = n_idx[row]
    S = jnp.zeros((_N, _K), dtype=score.dtype).at[col, scol].add(score * emask)
    CH = 250
    AS = jnp.concatenate([
        jax.ops.segment_sum(S[:, c:c + CH][col], row, num_segments=_N)
        for c in range(0, _K, CH)
    ], axis=1)
    Emat = _emat(S, AS)

    batch_new = jnp.zeros((_K,), dtype=jnp.int32)
    return x_new, Emat, batch_new, perm
